# Initial kernel scaffold; baseline (speedup 1.0000x reference)
import jax
import jax.numpy as jnp
from jax.experimental import pallas as pl


def kernel(x, Wg, W1, b1, W2, b2, head_w):
    raise NotImplementedError("write your pallas kernel here")


# speedup vs baseline: 18.6575x; 18.6575x over previous
import jax
import jax.numpy as jnp
from jax.experimental import pallas as pl
from jax.experimental.pallas import tpu as pltpu

N = 32768
DM, DH, E, TOPK, DD = 16, 8, 4, 2, 8
T = 4096
G = N // T


def _moe_kernel(x_ref, At_ref, b1_ref, W2rT_ref, b2T_ref, hwT_ref,
                out_ref, accs_ref, accP_ref, accf_ref):
    i = pl.program_id(0)
    xt = x_ref[...]
    At = At_ref[...]

    U = jnp.dot(At, xt, preferred_element_type=jnp.float32)
    logits = U[0:E, :]
    a = U[E:E + E * DH, :]

    eidx = jax.lax.broadcasted_iota(jnp.int32, (E, T), 0)
    m1 = jnp.max(logits, axis=0, keepdims=True)
    i1 = jnp.min(jnp.where(logits == m1, eidx, E), axis=0, keepdims=True)
    masked = jnp.where(eidx == i1, -jnp.inf, logits)
    m2 = jnp.max(masked, axis=0, keepdims=True)
    i2 = jnp.min(jnp.where(masked == m2, eidx, E), axis=0, keepdims=True)
    g1 = jax.nn.sigmoid(m1 - m2)
    g2 = jax.nn.sigmoid(m2 - m1)
    sel1 = eidx == i1
    sel2 = eidx == i2
    wmat = jnp.where(sel1, g1, 0.0) + jnp.where(sel2, g2, 0.0)
    cnt = sel1.astype(jnp.float32) + sel2.astype(jnp.float32)

    ex = jnp.exp(logits - m1)
    probs = ex / jnp.sum(ex, axis=0, keepdims=True)

    h = jnp.maximum(a + b1_ref[...], 0.0)
    r0 = jax.lax.broadcasted_iota(jnp.int32, (E * DH, E), 0) // DH
    r1 = jax.lax.broadcasted_iota(jnp.int32, (E * DH, E), 1)
    R = (r0 == r1).astype(jnp.float32)
    wrep = jnp.dot(R, wmat, preferred_element_type=jnp.float32)
    hw = h * wrep

    W2Ht = jnp.dot(hwT_ref[...], W2rT_ref[...],
                   preferred_element_type=jnp.float32)
    B2Ht = jnp.dot(hwT_ref[...], b2T_ref[...],
                   preferred_element_type=jnp.float32)
    z = (jnp.dot(W2Ht, hw, preferred_element_type=jnp.float32)
         + jnp.dot(B2Ht, wmat, preferred_element_type=jnp.float32))

    s_part = jnp.reshape(jnp.sum(z * z), (1, 1))
    P_part = jnp.sum(probs, axis=1, keepdims=True)
    f_part = jnp.sum(cnt, axis=1, keepdims=True)

    @pl.when(i == 0)
    def _():
        accs_ref[...] = s_part
        accP_ref[...] = P_part
        accf_ref[...] = f_part

    @pl.when(i > 0)
    def _():
        accs_ref[...] += s_part
        accP_ref[...] += P_part
        accf_ref[...] += f_part

    @pl.when(i == G - 1)
    def _():
        mean_z2 = accs_ref[...] / jnp.float32(N * DD)
        aux = jnp.reshape(
            jnp.float32(E) * jnp.sum(accP_ref[...] * accf_ref[...])
            / jnp.float32(N * TOPK) / jnp.float32(N), (1, 1))
        out_ref[...] = mean_z2 + aux


def kernel(x, Wg, W1, b1, W2, b2, head_w):
    xT = x.T
    At = jnp.concatenate(
        [Wg, jnp.transpose(W1, (1, 0, 2)).reshape(DM, E * DH)],
        axis=1).T
    b1c = b1.reshape(E * DH, 1)
    W2rT = W2.reshape(E * DH, DM).T
    b2T = b2.T
    head_wT = head_w.T

    out = pl.pallas_call(
        _moe_kernel,
        grid=(G,),
        in_specs=[
            pl.BlockSpec((DM, T), lambda i: (0, i)),
            pl.BlockSpec((E + E * DH, DM), lambda i: (0, 0)),
            pl.BlockSpec((E * DH, 1), lambda i: (0, 0)),
            pl.BlockSpec((DM, E * DH), lambda i: (0, 0)),
            pl.BlockSpec((DM, E), lambda i: (0, 0)),
            pl.BlockSpec((DD, DM), lambda i: (0, 0)),
        ],
        out_specs=pl.BlockSpec((1, 1), lambda i: (0, 0)),
        out_shape=jax.ShapeDtypeStruct((1, 1), jnp.float32),
        scratch_shapes=[
            pltpu.VMEM((1, 1), jnp.float32),
            pltpu.VMEM((E, 1), jnp.float32),
            pltpu.VMEM((E, 1), jnp.float32),
        ],
        compiler_params=pltpu.CompilerParams(
            dimension_semantics=("arbitrary",),
        ),
    )(xT, At, b1c, W2rT, b2T, head_wT)
    return out[0, 0]
